# baseline (speedup 1.0000x reference)
import functools

import jax
import jax.numpy as jnp
from jax import lax
from jax.experimental import pallas as pl
from jax.experimental.pallas import tpu as pltpu
from jax.experimental.pallas import tpu_sc as plsc

E = 8
K = 2
D = 2048
FF = 2048
T = 2048

BT = 128
ROUTED_PAD = K * T + E * BT
NB_R = ROUTED_PAD // BT
PT = ROUTED_PAD + 2 * T
SH_A = ROUTED_PAD
SH_B = ROUTED_PAD + T

NW = 32
G_CH = 16
C_CH = 4

FFT = 1024
NF = FF // FFT
BTS = 256


def _csum0(a, n):
    sh = 1
    while sh < n:
        a = a + jnp.concatenate(
            [jnp.zeros((sh, a.shape[1]), a.dtype), a[:-sh]], axis=0)
        sh *= 2
    return a


def _router_body(x_ref, wr_ref, w_ref, i_ref, r_ref, c_ref, carry_ref):
    blk = pl.program_id(0)

    @pl.when(blk == 0)
    def _():
        carry_ref[...] = jnp.zeros_like(carry_ref)

    x = x_ref[...]
    wr = wr_ref[...]
    logits = lax.dot_general(x, wr, (((1,), (0,)), ((), ())),
                             preferred_element_type=jnp.float32)
    iota = lax.broadcasted_iota(jnp.int32, logits.shape, 1)
    m1 = jnp.max(logits, axis=1, keepdims=True)
    i1 = jnp.min(jnp.where(logits == m1, iota, E), axis=1, keepdims=True)
    l2 = jnp.where(iota == i1, -1e30, logits)
    m2 = jnp.max(l2, axis=1, keepdims=True)
    i2 = jnp.min(jnp.where(l2 == m2, iota, E), axis=1, keepdims=True)
    e2 = jnp.exp(m2 - m1)
    w1 = 1.0 / (1.0 + e2)
    w2 = e2 / (1.0 + e2)
    oh0 = (iota == i1).astype(jnp.int32)
    oh1 = (iota == i2).astype(jnp.int32)
    p0 = _csum0(oh0, oh0.shape[0])
    p1 = _csum0(oh1, oh1.shape[0])
    t0v = p0[-1:]
    t1v = p1[-1:]
    carry = carry_ref[...]
    rank0 = jnp.sum((p0 - 1 + carry) * oh0, axis=1, keepdims=True)
    rank1 = jnp.sum((p1 - 1 + carry + t0v) * oh1, axis=1, keepdims=True)
    new_carry = carry + t0v + t1v
    carry_ref[...] = new_carry
    pad_f = jnp.zeros_like(logits[:, : E - 2])
    pad_i = jnp.zeros_like(iota[:, : E - 2])
    w_ref[...] = jnp.concatenate([w1, w2, pad_f], axis=1)
    i_ref[...] = jnp.concatenate([i1, i2, pad_i], axis=1)
    r_ref[...] = jnp.concatenate([rank0, rank1, pad_i], axis=1)
    c_ref[...] = new_carry


def _run_router(flat, Wr):
    bt = 256
    return pl.pallas_call(
        _router_body,
        grid=(T // bt,),
        in_specs=[
            pl.BlockSpec((bt, D), lambda i: (i, 0)),
            pl.BlockSpec((D, E), lambda i: (0, 0)),
        ],
        out_specs=[
            pl.BlockSpec((bt, E), lambda i: (i, 0)),
            pl.BlockSpec((bt, E), lambda i: (i, 0)),
            pl.BlockSpec((bt, E), lambda i: (i, 0)),
            pl.BlockSpec((1, E), lambda i: (0, 0)),
        ],
        out_shape=[
            jax.ShapeDtypeStruct((T, E), jnp.float32),
            jax.ShapeDtypeStruct((T, E), jnp.int32),
            jax.ShapeDtypeStruct((T, E), jnp.int32),
            jax.ShapeDtypeStruct((1, E), jnp.int32),
        ],
        scratch_shapes=[pltpu.VMEM((1, E), jnp.int32)],
    )(flat, Wr)


def _k1r_body(sc_ref, xs_ref, wg_ref, wu_ref, h_ref):
    b = pl.program_id(1)

    @pl.when(b < sc_ref[NB_R])
    def _():
        x = xs_ref[...]
        g = lax.dot_general(x, wg_ref[0], (((1,), (0,)), ((), ())),
                            preferred_element_type=jnp.float32)
        u = lax.dot_general(x, wu_ref[0], (((1,), (0,)), ((), ())),
                            preferred_element_type=jnp.float32)
        h_ref[...] = (g * lax.logistic(g) * u).astype(jnp.bfloat16)


def _run_k1r(sched, Xs, Wg, Wu):
    grid_spec = pltpu.PrefetchScalarGridSpec(
        num_scalar_prefetch=1,
        grid=(NF, NB_R),
        in_specs=[
            pl.BlockSpec((BT, D), lambda f, b, sc: (b, 0)),
            pl.BlockSpec((1, D, FFT), lambda f, b, sc: (sc[b], 0, f)),
            pl.BlockSpec((1, D, FFT), lambda f, b, sc: (sc[b], 0, f)),
        ],
        out_specs=pl.BlockSpec((BT, FFT), lambda f, b, sc: (b, f)),
    )
    return pl.pallas_call(
        _k1r_body,
        grid_spec=grid_spec,
        out_shape=jax.ShapeDtypeStruct((ROUTED_PAD, FF), jnp.bfloat16),
    )(sched, Xs, Wg, Wu)


def _k2r_body(sc_ref, h_ref, wd_ref, g_ref, o_ref, wdbf_ref):
    b = pl.program_id(0)
    active = b < sc_ref[NB_R]
    changed = jnp.logical_or(
        b == 0, sc_ref[jnp.maximum(b, 1) - 1] != sc_ref[b])

    @pl.when(jnp.logical_and(active, changed))
    def _():
        wdbf_ref[...] = wd_ref[0].astype(jnp.bfloat16)

    @pl.when(active)
    def _():
        o = lax.dot_general(h_ref[...], wdbf_ref[...],
                            (((1,), (0,)), ((), ())),
                            preferred_element_type=jnp.float32)
        o_ref[...] = o * g_ref[...][:, :1]


def _run_k2r(sched, H, Wd, gates_pad):
    grid_spec = pltpu.PrefetchScalarGridSpec(
        num_scalar_prefetch=1,
        grid=(NB_R,),
        in_specs=[
            pl.BlockSpec((BT, FF), lambda b, sc: (b, 0)),
            pl.BlockSpec((1, FF, D), lambda b, sc: (sc[b], 0, 0)),
            pl.BlockSpec((BT, 128), lambda b, sc: (b, 0)),
        ],
        out_specs=pl.BlockSpec((BT, D), lambda b, sc: (b, 0)),
        scratch_shapes=[pltpu.VMEM((FF, D), jnp.bfloat16)],
    )
    return pl.pallas_call(
        _k2r_body,
        grid_spec=grid_spec,
        out_shape=jax.ShapeDtypeStruct((PT, D), jnp.float32),
    )(sched, H, Wd, gates_pad)


def _k1s_body(x_ref, wg_ref, wu_ref, h_ref):
    x = x_ref[...]
    g = lax.dot_general(x, wg_ref[...], (((1,), (0,)), ((), ())),
                        preferred_element_type=jnp.float32)
    u = lax.dot_general(x, wu_ref[...], (((1,), (0,)), ((), ())),
                        preferred_element_type=jnp.float32)
    h_ref[...] = (g * lax.logistic(g) * u).astype(jnp.bfloat16)


def _run_k1s(flat, Wsg, Wsu):
    nfs = 2 * FF // FFT
    return pl.pallas_call(
        _k1s_body,
        grid=(nfs, T // BTS),
        in_specs=[
            pl.BlockSpec((BTS, D), lambda f, b: (b, 0)),
            pl.BlockSpec((D, FFT), lambda f, b: (0, f)),
            pl.BlockSpec((D, FFT), lambda f, b: (0, f)),
        ],
        out_specs=pl.BlockSpec((BTS, FFT), lambda f, b: (b, f)),
        out_shape=jax.ShapeDtypeStruct((T, 2 * FF), jnp.bfloat16),
    )(flat, Wsg, Wsu)


def _k2s_body(prev_ref, h_ref, wd_ref, o_ref):
    del prev_ref
    h = h_ref[...].astype(jnp.float32)
    o_ref[...] = lax.dot_general(h, wd_ref[...],
                                 (((1,), (0,)), ((), ())),
                                 preferred_element_type=jnp.float32)


def _run_k2s(out1, Hs, Wsd):
    return pl.pallas_call(
        _k2s_body,
        grid=(2, T // BTS),
        in_specs=[
            pl.BlockSpec(memory_space=pl.ANY),
            pl.BlockSpec((BTS, FF), lambda f, b: (b, f)),
            pl.BlockSpec((FF, D), lambda f, b: (f, 0)),
        ],
        out_specs=pl.BlockSpec(
            (BTS, D), lambda f, b: (ROUTED_PAD // BTS + f * (T // BTS) + b, 0)),
        out_shape=jax.ShapeDtypeStruct((PT, D), jnp.float32),
        input_output_aliases={0: 0},
    )(out1, Hs, Wsd)


@functools.cache
def _sc_dispatch_kernel():
    mesh = plsc.VectorSubcoreMesh(core_axis_name="c", subcore_axis_name="s")
    tok_per_w = T // NW
    n_ch = tok_per_w // G_CH
    spw = K * tok_per_w

    @functools.partial(
        pl.kernel,
        out_type=(jax.ShapeDtypeStruct((ROUTED_PAD, D), jnp.float32),
                  jax.ShapeDtypeStruct((ROUTED_PAD, 128), jnp.float32)),
        mesh=mesh,
        scratch_types=[
            pltpu.VMEM((n_ch, G_CH), jnp.int32),
            pltpu.VMEM((n_ch, G_CH), jnp.int32),
            pltpu.VMEM((spw,), jnp.int32),
            pltpu.VMEM((spw, 128), jnp.float32),
            pltpu.VMEM((2, G_CH, D), jnp.float32),
            pltpu.SemaphoreType.DMA,
            pltpu.SemaphoreType.DMA,
            pltpu.SemaphoreType.DMA,
            pltpu.SemaphoreType.DMA,
            pltpu.SemaphoreType.DMA,
            pltpu.SemaphoreType.DMA,
            pltpu.SemaphoreType.DMA,
        ],
    )
    def k(flat_hbm, p0_hbm, p1_hbm, ps_hbm, g16_hbm, xs_hbm, go_hbm,
          p0_v, p1_v, ps_v, g_v, slab, l0, l1, s00, s01, s10, s11, gs):
        wid = lax.axis_index("s") * 2 + lax.axis_index("c")
        pltpu.sync_copy(p0_hbm.at[pl.ds(wid * n_ch, n_ch)], p0_v)
        pltpu.sync_copy(p1_hbm.at[pl.ds(wid * n_ch, n_ch)], p1_v)
        pltpu.sync_copy(ps_hbm.at[wid], ps_v)
        pltpu.sync_copy(g16_hbm.at[pl.ds(wid * spw, spw)], g_v)
        gd = pltpu.async_copy(g_v, go_hbm.at[ps_v], gs)
        lsem = (l0, l1)
        s0sem = (s00, s01)
        s1sem = (s10, s11)
        ld = [None, None]
        s0 = [None, None]
        s1 = [None, None]
        for c in range(n_ch):
            bb = c & 1
            if c >= 2:
                s0[bb].wait()
                s1[bb].wait()
            ld[bb] = pltpu.async_copy(
                flat_hbm.at[pl.ds(wid * tok_per_w + c * G_CH, G_CH)],
                slab.at[bb], lsem[bb])
            if c >= 1:
                p = (c - 1) & 1
                ld[p].wait()
                s0[p] = pltpu.async_copy(
                    slab.at[p], xs_hbm.at[p0_v.at[c - 1]], s0sem[p])
                s1[p] = pltpu.async_copy(
                    slab.at[p], xs_hbm.at[p1_v.at[c - 1]], s1sem[p])
        p = (n_ch - 1) & 1
        ld[p].wait()
        s0[p] = pltpu.async_copy(slab.at[p], xs_hbm.at[p0_v.at[n_ch - 1]],
                                 s0sem[p])
        s1[p] = pltpu.async_copy(slab.at[p], xs_hbm.at[p1_v.at[n_ch - 1]],
                                 s1sem[p])
        if n_ch >= 2:
            q = (n_ch - 2) & 1
            s0[q].wait()
            s1[q].wait()
        s0[p].wait()
        s1[p].wait()
        gd.wait()

    return k


@functools.cache
def _sc_combine_kernel():
    mesh = plsc.VectorSubcoreMesh(core_axis_name="c", subcore_axis_name="s")
    tok_per_w = T // NW
    n_ch = tok_per_w // C_CH
    rpc = 4 * C_CH

    nbuf = 3
    rg = K * C_CH

    @functools.partial(
        pl.kernel,
        out_type=jax.ShapeDtypeStruct((T, D), jnp.float32),
        mesh=mesh,
        scratch_types=[
            pltpu.VMEM((n_ch * rg,), jnp.int32),
            pltpu.VMEM((nbuf, rg, D), jnp.float32),
            pltpu.VMEM((nbuf, C_CH, D), jnp.float32),
            pltpu.VMEM((nbuf, C_CH, D), jnp.float32),
            pltpu.VMEM((nbuf, C_CH, D), jnp.float32),
            pltpu.SemaphoreType.DMA,
            pltpu.SemaphoreType.DMA,
            pltpu.SemaphoreType.DMA,
            pltpu.SemaphoreType.DMA,
            pltpu.SemaphoreType.DMA,
            pltpu.SemaphoreType.DMA,
            pltpu.SemaphoreType.DMA,
            pltpu.SemaphoreType.DMA,
            pltpu.SemaphoreType.DMA,
            pltpu.SemaphoreType.DMA,
            pltpu.SemaphoreType.DMA,
            pltpu.SemaphoreType.DMA,
        ],
    )
    def k(rows_hbm, idx_hbm, out_hbm, idx_v, bufr, bufa, bufb, obuf,
          g0, g1, g2, a0, a1, a2, b0, b1, b2, w0, w1, w2):
        wid = lax.axis_index("s") * 2 + lax.axis_index("c")
        tbase = wid * tok_per_w
        pltpu.sync_copy(idx_hbm.at[pl.ds(wid * n_ch * rg, n_ch * rg)], idx_v)
        gsem = (g0, g1, g2)
        asem = (a0, a1, a2)
        bsem = (b0, b1, b2)
        wsem = (w0, w1, w2)
        gd = [None] * nbuf
        ad = [None] * nbuf
        bd = [None] * nbuf
        wd_ = [None] * nbuf

        def compute(p):
            def col(kk, _):
                sl = pl.ds(kk * 16, 16)
                for i in range(C_CH):
                    obuf[p, i, sl] = (bufr[p, i, sl]
                                      + bufr[p, C_CH + i, sl]
                                      + bufa[p, i, sl]
                                      + bufb[p, i, sl])
                return ()
            lax.fori_loop(0, D // 16, col, ())

        for c in range(n_ch + nbuf - 1):
            if c < n_ch:
                bb = c % nbuf
                if c >= nbuf:
                    wd_[bb].wait()
                t0 = tbase + c * C_CH
                gd[bb] = pltpu.async_copy(
                    rows_hbm.at[idx_v.at[pl.ds(c * rg, rg)]],
                    bufr.at[bb], gsem[bb])
                ad[bb] = pltpu.async_copy(
                    rows_hbm.at[pl.ds(SH_A + t0, C_CH)], bufa.at[bb], asem[bb])
                bd[bb] = pltpu.async_copy(
                    rows_hbm.at[pl.ds(SH_B + t0, C_CH)], bufb.at[bb], bsem[bb])
            j = c - (nbuf - 1)
            if j >= 0:
                p = j % nbuf
                gd[p].wait()
                ad[p].wait()
                bd[p].wait()
                compute(p)
                wd_[p] = pltpu.async_copy(
                    obuf.at[p],
                    out_hbm.at[pl.ds(tbase + j * C_CH, C_CH)], wsem[p])
        for j in range(max(0, n_ch - nbuf), n_ch):
            wd_[j % nbuf].wait()

    return k


def _build_schedule(idx, rank, wts, counts):
    pc = ((counts + BT - 1) // BT) * BT
    cpc = jnp.cumsum(pc)
    poff = jnp.concatenate([jnp.zeros(1, cpc.dtype), cpc])
    pos_pair = (poff[idx] + rank).astype(jnp.int32)

    p0_2d = pos_pair[:, 0].reshape(T // G_CH, G_CH)
    p1_2d = pos_pair[:, 1].reshape(T // G_CH, G_CH)
    ps_2d = pos_pair.reshape(NW, K * T // NW)
    gates16 = jnp.broadcast_to(
        wts.reshape(T * K, 1), (T * K, 128)).astype(jnp.float32)

    bstart = jnp.arange(NB_R) * BT
    be_r = jnp.clip(jnp.searchsorted(poff, bstart, side="right") - 1, 0, E - 1)
    nact = (cpc[-1] // BT).astype(jnp.int32)
    sched = jnp.concatenate([be_r.astype(jnp.int32), nact[None]])

    idx_comb = jnp.concatenate(
        [pos_pair[:, 0].reshape(-1, C_CH), pos_pair[:, 1].reshape(-1, C_CH)],
        axis=1).reshape(-1)
    return p0_2d, p1_2d, ps_2d, gates16, sched, idx_comb


def kernel(hidden_states, Wr, Wsg, Wsu, Wsd, Wg, Wu, Wd):
    b, s, d = hidden_states.shape
    flat = hidden_states.reshape(-1, d)

    w8, i8, r8, c8 = _run_router(flat, Wr)
    p0_2d, p1_2d, ps_2d, gates16, sched, idx_comb = _build_schedule(
        i8[:, :K], r8[:, :K], w8[:, :K], c8[0])

    Xs, gates_out = _sc_dispatch_kernel()(flat, p0_2d, p1_2d, ps_2d, gates16)
    Hr = _run_k1r(sched, Xs, Wg, Wu)
    Out1 = _run_k2r(sched, Hr, Wd, gates_out)
    Hs = _run_k1s(flat, Wsg, Wsu)
    Out2 = _run_k2s(Out1, Hs, Wsd)
    out = _sc_combine_kernel()(Out2, idx_comb)
    return out.reshape(b, s, d)


# speedup vs baseline: 1.0113x; 1.0113x over previous
import functools

import jax
import jax.numpy as jnp
from jax import lax
from jax.experimental import pallas as pl
from jax.experimental.pallas import tpu as pltpu
from jax.experimental.pallas import tpu_sc as plsc

E = 8
K = 2
D = 2048
FF = 2048
T = 2048

BT = 128
ROUTED_PAD = K * T + E * BT
NB_R = ROUTED_PAD // BT
PT = ROUTED_PAD + 2 * T
SH_A = ROUTED_PAD
SH_B = ROUTED_PAD + T

NW = 32
G_CH = 16
C_CH = 4

FFT = 1024
NF = FF // FFT
BTS = 256


def _csum0(a, n):
    sh = 1
    while sh < n:
        a = a + jnp.concatenate(
            [jnp.zeros((sh, a.shape[1]), a.dtype), a[:-sh]], axis=0)
        sh *= 2
    return a


def _router_body(x_ref, wr_ref, w_ref, i_ref, r_ref, c_ref, carry_ref):
    blk = pl.program_id(0)

    @pl.when(blk == 0)
    def _():
        carry_ref[...] = jnp.zeros_like(carry_ref)

    x = x_ref[...]
    wr = wr_ref[...]
    logits = lax.dot_general(x, wr, (((1,), (0,)), ((), ())),
                             preferred_element_type=jnp.float32)
    iota = lax.broadcasted_iota(jnp.int32, logits.shape, 1)
    m1 = jnp.max(logits, axis=1, keepdims=True)
    i1 = jnp.min(jnp.where(logits == m1, iota, E), axis=1, keepdims=True)
    l2 = jnp.where(iota == i1, -1e30, logits)
    m2 = jnp.max(l2, axis=1, keepdims=True)
    i2 = jnp.min(jnp.where(l2 == m2, iota, E), axis=1, keepdims=True)
    e2 = jnp.exp(m2 - m1)
    w1 = 1.0 / (1.0 + e2)
    w2 = e2 / (1.0 + e2)
    oh0 = (iota == i1).astype(jnp.int32)
    oh1 = (iota == i2).astype(jnp.int32)
    p0 = _csum0(oh0, oh0.shape[0])
    p1 = _csum0(oh1, oh1.shape[0])
    t0v = p0[-1:]
    t1v = p1[-1:]
    carry = carry_ref[...]
    rank0 = jnp.sum((p0 - 1 + carry) * oh0, axis=1, keepdims=True)
    rank1 = jnp.sum((p1 - 1 + carry + t0v) * oh1, axis=1, keepdims=True)
    new_carry = carry + t0v + t1v
    carry_ref[...] = new_carry
    pad_f = jnp.zeros_like(logits[:, : E - 2])
    pad_i = jnp.zeros_like(iota[:, : E - 2])
    w_ref[...] = jnp.concatenate([w1, w2, pad_f], axis=1)
    i_ref[...] = jnp.concatenate([i1, i2, pad_i], axis=1)
    r_ref[...] = jnp.concatenate([rank0, rank1, pad_i], axis=1)
    c_ref[...] = new_carry


def _run_router(flat, Wr):
    bt = 256
    return pl.pallas_call(
        _router_body,
        grid=(T // bt,),
        in_specs=[
            pl.BlockSpec((bt, D), lambda i: (i, 0)),
            pl.BlockSpec((D, E), lambda i: (0, 0)),
        ],
        out_specs=[
            pl.BlockSpec((bt, E), lambda i: (i, 0)),
            pl.BlockSpec((bt, E), lambda i: (i, 0)),
            pl.BlockSpec((bt, E), lambda i: (i, 0)),
            pl.BlockSpec((1, E), lambda i: (0, 0)),
        ],
        out_shape=[
            jax.ShapeDtypeStruct((T, E), jnp.float32),
            jax.ShapeDtypeStruct((T, E), jnp.int32),
            jax.ShapeDtypeStruct((T, E), jnp.int32),
            jax.ShapeDtypeStruct((1, E), jnp.int32),
        ],
        scratch_shapes=[pltpu.VMEM((1, E), jnp.int32)],
    )(flat, Wr)


def _k1r_body(sc_ref, xs_ref, wg_ref, wu_ref, h_ref):
    b = pl.program_id(1)

    @pl.when(b < sc_ref[NB_R])
    def _():
        x = xs_ref[...]
        g = lax.dot_general(x, wg_ref[0], (((1,), (0,)), ((), ())),
                            preferred_element_type=jnp.float32)
        u = lax.dot_general(x, wu_ref[0], (((1,), (0,)), ((), ())),
                            preferred_element_type=jnp.float32)
        h_ref[...] = (g * lax.logistic(g) * u).astype(jnp.bfloat16)


def _run_k1r(sched, Xs, Wg, Wu):
    grid_spec = pltpu.PrefetchScalarGridSpec(
        num_scalar_prefetch=1,
        grid=(NF, NB_R),
        in_specs=[
            pl.BlockSpec((BT, D), lambda f, b, sc: (b, 0)),
            pl.BlockSpec((1, D, FFT), lambda f, b, sc: (sc[b], 0, f)),
            pl.BlockSpec((1, D, FFT), lambda f, b, sc: (sc[b], 0, f)),
        ],
        out_specs=pl.BlockSpec((BT, FFT), lambda f, b, sc: (b, f)),
    )
    return pl.pallas_call(
        _k1r_body,
        grid_spec=grid_spec,
        out_shape=jax.ShapeDtypeStruct((ROUTED_PAD, FF), jnp.bfloat16),
    )(sched, Xs, Wg, Wu)


def _k2r_body(sc_ref, h_ref, wd_ref, g_ref, o_ref):
    b = pl.program_id(0)

    @pl.when(b < sc_ref[NB_R])
    def _():
        h = h_ref[...].astype(jnp.float32)
        o = lax.dot_general(h, wd_ref[0], (((1,), (0,)), ((), ())),
                            preferred_element_type=jnp.float32)
        o_ref[...] = o * g_ref[...][:, :1]


def _run_k2r(sched, H, Wd, gates_pad):
    grid_spec = pltpu.PrefetchScalarGridSpec(
        num_scalar_prefetch=1,
        grid=(NB_R,),
        in_specs=[
            pl.BlockSpec((BT, FF), lambda b, sc: (b, 0)),
            pl.BlockSpec((1, FF, D), lambda b, sc: (sc[b], 0, 0)),
            pl.BlockSpec((BT, 128), lambda b, sc: (b, 0)),
        ],
        out_specs=pl.BlockSpec((BT, D), lambda b, sc: (b, 0)),
    )
    return pl.pallas_call(
        _k2r_body,
        grid_spec=grid_spec,
        out_shape=jax.ShapeDtypeStruct((PT, D), jnp.float32),
    )(sched, H, Wd, gates_pad)


def _k1s_body(x_ref, wg_ref, wu_ref, h_ref):
    x = x_ref[...]
    g = lax.dot_general(x, wg_ref[...], (((1,), (0,)), ((), ())),
                        preferred_element_type=jnp.float32)
    u = lax.dot_general(x, wu_ref[...], (((1,), (0,)), ((), ())),
                        preferred_element_type=jnp.float32)
    h_ref[...] = (g * lax.logistic(g) * u).astype(jnp.bfloat16)


def _run_k1s(flat, Wsg, Wsu):
    nfs = 2 * FF // FFT
    return pl.pallas_call(
        _k1s_body,
        grid=(nfs, T // BTS),
        in_specs=[
            pl.BlockSpec((BTS, D), lambda f, b: (b, 0)),
            pl.BlockSpec((D, FFT), lambda f, b: (0, f)),
            pl.BlockSpec((D, FFT), lambda f, b: (0, f)),
        ],
        out_specs=pl.BlockSpec((BTS, FFT), lambda f, b: (b, f)),
        out_shape=jax.ShapeDtypeStruct((T, 2 * FF), jnp.bfloat16),
    )(flat, Wsg, Wsu)


def _k2s_body(prev_ref, h_ref, wd_ref, o_ref):
    del prev_ref
    h = h_ref[...].astype(jnp.float32)
    o_ref[...] = lax.dot_general(h, wd_ref[...],
                                 (((1,), (0,)), ((), ())),
                                 preferred_element_type=jnp.float32)


def _run_k2s(out1, Hs, Wsd):
    return pl.pallas_call(
        _k2s_body,
        grid=(2, T // BTS),
        in_specs=[
            pl.BlockSpec(memory_space=pl.ANY),
            pl.BlockSpec((BTS, FF), lambda f, b: (b, f)),
            pl.BlockSpec((FF, D), lambda f, b: (f, 0)),
        ],
        out_specs=pl.BlockSpec(
            (BTS, D), lambda f, b: (ROUTED_PAD // BTS + f * (T // BTS) + b, 0)),
        out_shape=jax.ShapeDtypeStruct((PT, D), jnp.float32),
        input_output_aliases={0: 0},
    )(out1, Hs, Wsd)


@functools.cache
def _sc_dispatch_kernel():
    mesh = plsc.VectorSubcoreMesh(core_axis_name="c", subcore_axis_name="s")
    tok_per_w = T // NW
    n_ch = tok_per_w // G_CH
    spw = K * tok_per_w

    @functools.partial(
        pl.kernel,
        out_type=(jax.ShapeDtypeStruct((ROUTED_PAD, D), jnp.float32),
                  jax.ShapeDtypeStruct((ROUTED_PAD, 128), jnp.float32)),
        mesh=mesh,
        scratch_types=[
            pltpu.VMEM((n_ch, G_CH), jnp.int32),
            pltpu.VMEM((n_ch, G_CH), jnp.int32),
            pltpu.VMEM((spw,), jnp.int32),
            pltpu.VMEM((spw, 128), jnp.float32),
            pltpu.VMEM((2, G_CH, D), jnp.float32),
            pltpu.SemaphoreType.DMA,
            pltpu.SemaphoreType.DMA,
            pltpu.SemaphoreType.DMA,
            pltpu.SemaphoreType.DMA,
            pltpu.SemaphoreType.DMA,
            pltpu.SemaphoreType.DMA,
            pltpu.SemaphoreType.DMA,
        ],
    )
    def k(flat_hbm, p0_hbm, p1_hbm, ps_hbm, g16_hbm, xs_hbm, go_hbm,
          p0_v, p1_v, ps_v, g_v, slab, l0, l1, s00, s01, s10, s11, gs):
        wid = lax.axis_index("s") * 2 + lax.axis_index("c")
        pltpu.sync_copy(p0_hbm.at[pl.ds(wid * n_ch, n_ch)], p0_v)
        pltpu.sync_copy(p1_hbm.at[pl.ds(wid * n_ch, n_ch)], p1_v)
        pltpu.sync_copy(ps_hbm.at[wid], ps_v)
        pltpu.sync_copy(g16_hbm.at[pl.ds(wid * spw, spw)], g_v)
        gd = pltpu.async_copy(g_v, go_hbm.at[ps_v], gs)
        lsem = (l0, l1)
        s0sem = (s00, s01)
        s1sem = (s10, s11)
        ld = [None, None]
        s0 = [None, None]
        s1 = [None, None]
        for c in range(n_ch):
            bb = c & 1
            if c >= 2:
                s0[bb].wait()
                s1[bb].wait()
            ld[bb] = pltpu.async_copy(
                flat_hbm.at[pl.ds(wid * tok_per_w + c * G_CH, G_CH)],
                slab.at[bb], lsem[bb])
            if c >= 1:
                p = (c - 1) & 1
                ld[p].wait()
                s0[p] = pltpu.async_copy(
                    slab.at[p], xs_hbm.at[p0_v.at[c - 1]], s0sem[p])
                s1[p] = pltpu.async_copy(
                    slab.at[p], xs_hbm.at[p1_v.at[c - 1]], s1sem[p])
        p = (n_ch - 1) & 1
        ld[p].wait()
        s0[p] = pltpu.async_copy(slab.at[p], xs_hbm.at[p0_v.at[n_ch - 1]],
                                 s0sem[p])
        s1[p] = pltpu.async_copy(slab.at[p], xs_hbm.at[p1_v.at[n_ch - 1]],
                                 s1sem[p])
        if n_ch >= 2:
            q = (n_ch - 2) & 1
            s0[q].wait()
            s1[q].wait()
        s0[p].wait()
        s1[p].wait()
        gd.wait()

    return k


@functools.cache
def _sc_combine_kernel():
    mesh = plsc.VectorSubcoreMesh(core_axis_name="c", subcore_axis_name="s")
    tok_per_w = T // NW
    n_ch = tok_per_w // C_CH

    nbuf = 3
    rg = K * C_CH

    @functools.partial(
        pl.kernel,
        out_type=jax.ShapeDtypeStruct((T, D), jnp.float32),
        mesh=mesh,
        scratch_types=[
            pltpu.VMEM((n_ch * rg,), jnp.int32),
            pltpu.VMEM((nbuf, rg, D), jnp.float32),
            pltpu.VMEM((nbuf, C_CH, D), jnp.float32),
            pltpu.VMEM((nbuf, C_CH, D), jnp.float32),
            pltpu.VMEM((nbuf, C_CH, D), jnp.float32),
            pltpu.SemaphoreType.DMA,
            pltpu.SemaphoreType.DMA,
            pltpu.SemaphoreType.DMA,
            pltpu.SemaphoreType.DMA,
            pltpu.SemaphoreType.DMA,
            pltpu.SemaphoreType.DMA,
            pltpu.SemaphoreType.DMA,
            pltpu.SemaphoreType.DMA,
            pltpu.SemaphoreType.DMA,
            pltpu.SemaphoreType.DMA,
            pltpu.SemaphoreType.DMA,
            pltpu.SemaphoreType.DMA,
        ],
    )
    def k(rows_hbm, idx_hbm, out_hbm, idx_v, bufr, bufa, bufb, obuf,
          g0, g1, g2, a0, a1, a2, b0, b1, b2, w0, w1, w2):
        wid = lax.axis_index("s") * 2 + lax.axis_index("c")
        tbase = wid * tok_per_w
        pltpu.sync_copy(idx_hbm.at[pl.ds(wid * n_ch * rg, n_ch * rg)], idx_v)
        gsem = (g0, g1, g2)
        asem = (a0, a1, a2)
        bsem = (b0, b1, b2)
        wsem = (w0, w1, w2)
        gd = [None] * nbuf
        ad = [None] * nbuf
        bd = [None] * nbuf
        wd_ = [None] * nbuf

        def compute(p):
            def col(kk, _):
                sl = pl.ds(kk * 16, 16)
                for i in range(C_CH):
                    obuf[p, i, sl] = (bufr[p, i, sl]
                                      + bufr[p, C_CH + i, sl]
                                      + bufa[p, i, sl]
                                      + bufb[p, i, sl])
                return ()
            lax.fori_loop(0, D // 16, col, ())

        for c in range(n_ch + nbuf - 1):
            if c < n_ch:
                bb = c % nbuf
                if c >= nbuf:
                    wd_[bb].wait()
                t0 = tbase + c * C_CH
                gd[bb] = pltpu.async_copy(
                    rows_hbm.at[idx_v.at[pl.ds(c * rg, rg)]],
                    bufr.at[bb], gsem[bb])
                ad[bb] = pltpu.async_copy(
                    rows_hbm.at[pl.ds(SH_A + t0, C_CH)], bufa.at[bb], asem[bb])
                bd[bb] = pltpu.async_copy(
                    rows_hbm.at[pl.ds(SH_B + t0, C_CH)], bufb.at[bb], bsem[bb])
            j = c - (nbuf - 1)
            if j >= 0:
                p = j % nbuf
                gd[p].wait()
                ad[p].wait()
                bd[p].wait()
                compute(p)
                wd_[p] = pltpu.async_copy(
                    obuf.at[p],
                    out_hbm.at[pl.ds(tbase + j * C_CH, C_CH)], wsem[p])
        for j in range(max(0, n_ch - nbuf), n_ch):
            wd_[j % nbuf].wait()

    return k


def _build_schedule(idx, rank, wts, counts):
    pc = ((counts + BT - 1) // BT) * BT
    cpc = jnp.cumsum(pc)
    poff = jnp.concatenate([jnp.zeros(1, cpc.dtype), cpc])
    pos_pair = (poff[idx] + rank).astype(jnp.int32)

    p0_2d = pos_pair[:, 0].reshape(T // G_CH, G_CH)
    p1_2d = pos_pair[:, 1].reshape(T // G_CH, G_CH)
    ps_2d = pos_pair.reshape(NW, K * T // NW)
    gates16 = jnp.broadcast_to(
        wts.reshape(T * K, 1), (T * K, 128)).astype(jnp.float32)

    bstart = jnp.arange(NB_R) * BT
    be_r = jnp.clip(jnp.searchsorted(poff, bstart, side="right") - 1, 0, E - 1)
    nact = (cpc[-1] // BT).astype(jnp.int32)
    sched = jnp.concatenate([be_r.astype(jnp.int32), nact[None]])

    idx_comb = jnp.concatenate(
        [pos_pair[:, 0].reshape(-1, C_CH), pos_pair[:, 1].reshape(-1, C_CH)],
        axis=1).reshape(-1)
    return p0_2d, p1_2d, ps_2d, gates16, sched, idx_comb


def kernel(hidden_states, Wr, Wsg, Wsu, Wsd, Wg, Wu, Wd):
    b, s, d = hidden_states.shape
    flat = hidden_states.reshape(-1, d)

    w8, i8, r8, c8 = _run_router(flat, Wr)
    p0_2d, p1_2d, ps_2d, gates16, sched, idx_comb = _build_schedule(
        i8[:, :K], r8[:, :K], w8[:, :K], c8[0])

    Xs, gates_out = _sc_dispatch_kernel()(flat, p0_2d, p1_2d, ps_2d, gates16)
    Hr = _run_k1r(sched, Xs, Wg, Wu)
    Out1 = _run_k2r(sched, Hr, Wd, gates_out)
    Hs = _run_k1s(flat, Wsg, Wsu)
    Out2 = _run_k2s(Out1, Hs, Wsd)
    out = _sc_combine_kernel()(Out2, idx_comb)
    return out.reshape(b, s, d)
